# Initial kernel scaffold; baseline (speedup 1.0000x reference)
import jax
import jax.numpy as jnp
from jax.experimental import pallas as pl


def kernel(text, emb_weight, fc_w, fc_b):
    raise NotImplementedError("write your pallas kernel here")


# speedup vs baseline: 1.6881x; 1.6881x over previous
import functools

import jax
import jax.numpy as jnp
from jax import lax
from jax.experimental import pallas as pl
from jax.experimental.pallas import tpu as pltpu
from jax.experimental.pallas import tpu_sc as plsc

B = 16384
L = 50
D = 64
C = 20

NW = 32
BPW = B // NW
CHUNK = 8
NCHUNK = BPW // CHUNK

_mesh = plsc.VectorSubcoreMesh(core_axis_name="c", subcore_axis_name="s")


@functools.partial(
    pl.kernel,
    mesh=_mesh,
    out_type=jax.ShapeDtypeStruct((B, D), jnp.float32),
    scratch_types=[
        pltpu.VMEM((CHUNK, L), jnp.int32),
        pltpu.VMEM((CHUNK, L, D), jnp.float32),
        pltpu.VMEM((CHUNK, D), jnp.float32),
        pltpu.SemaphoreType.DMA,
    ],
    compiler_params=pltpu.CompilerParams(use_tc_tiling_on_sc=False),
)
def _sc_embed(text_ref, table_ref, out_ref, idx_v, rows_v, mean_v, gsem):
    wid = lax.axis_index("s") * 2 + lax.axis_index("c")
    base = wid * BPW

    def chunk_body(ci, carry):
        bag0 = base + ci * CHUNK
        pltpu.sync_copy(text_ref.at[pl.ds(bag0, CHUNK), :], idx_v)
        copies = [
            pltpu.async_copy(table_ref.at[idx_v.at[b]], rows_v.at[b], gsem)
            for b in range(CHUNK)
        ]
        for cp in copies:
            cp.wait()
        for b in range(CHUNK):
            for v in range(D // 16):
                def l_body(l, acc):
                    return acc + rows_v[b, l, pl.ds(v * 16, 16)]
                acc = lax.fori_loop(0, L, l_body, jnp.zeros((16,), jnp.float32))
                mean_v[b, pl.ds(v * 16, 16)] = acc * (1.0 / L)
        pltpu.sync_copy(mean_v, out_ref.at[pl.ds(bag0, CHUNK), :])
        return carry

    lax.fori_loop(0, NCHUNK, chunk_body, 0)


def _fc_body(x_ref, w_ref, b_ref, o_ref):
    o_ref[...] = lax.dot_general(
        x_ref[...], w_ref[...],
        dimension_numbers=(((1,), (1,)), ((), ())),
        preferred_element_type=jnp.float32,
    ) + b_ref[...]


def _fc(x, w, b2d):
    bm = 1024
    return pl.pallas_call(
        _fc_body,
        grid=(B // bm,),
        in_specs=[
            pl.BlockSpec((bm, D), lambda i: (i, 0)),
            pl.BlockSpec((C, D), lambda i: (0, 0)),
            pl.BlockSpec((1, C), lambda i: (0, 0)),
        ],
        out_specs=pl.BlockSpec((bm, C), lambda i: (i, 0)),
        out_shape=jax.ShapeDtypeStruct((B, C), jnp.float32),
    )(x, w, b2d)


def kernel(text, emb_weight, fc_w, fc_b):
    pooled = _sc_embed(text.astype(jnp.int32), emb_weight)
    return _fc(pooled, fc_w, fc_b.reshape(1, C))


# speedup vs baseline: 2.7365x; 1.6211x over previous
import functools

import jax
import jax.numpy as jnp
from jax import lax
from jax.experimental import pallas as pl
from jax.experimental.pallas import tpu as pltpu
from jax.experimental.pallas import tpu_sc as plsc

B = 16384
L = 50
D = 64
C = 20

NW = 32
BPW = B // NW
CHUNK = 8
NCHUNK = BPW // CHUNK
NV = D // 16

_mesh = plsc.VectorSubcoreMesh(core_axis_name="c", subcore_axis_name="s")


@functools.partial(
    pl.kernel,
    mesh=_mesh,
    out_type=jax.ShapeDtypeStruct((B, D), jnp.float32),
    scratch_types=[
        pltpu.VMEM((CHUNK, L), jnp.int32),
        pltpu.VMEM((CHUNK, L), jnp.int32),
        pltpu.VMEM((CHUNK * L, D), jnp.float32),
        pltpu.VMEM((CHUNK * L, D), jnp.float32),
        pltpu.VMEM((CHUNK, D), jnp.float32),
        pltpu.VMEM((CHUNK, D), jnp.float32),
        pltpu.SemaphoreType.DMA,
        pltpu.SemaphoreType.DMA,
        pltpu.SemaphoreType.DMA,
        pltpu.SemaphoreType.DMA,
        pltpu.SemaphoreType.DMA,
        pltpu.SemaphoreType.DMA,
    ],
    compiler_params=pltpu.CompilerParams(use_tc_tiling_on_sc=False),
)
def _sc_embed(text_ref, table_ref, out_ref,
              idx0, idx1, rows0, rows1, mean0, mean1,
              sidx0, sidx1, srows0, srows1, sout0, sout1):
    idx = (idx0, idx1)
    rows = (rows0, rows1)
    mean = (mean0, mean1)
    sidx = (sidx0, sidx1)
    srows = (srows0, srows1)
    sout = (sout0, sout1)

    wid = lax.axis_index("s") * 2 + lax.axis_index("c")
    base = wid * BPW

    def fire_gathers(ci, par):
        for b in range(CHUNK):
            pltpu.async_copy(
                table_ref.at[idx[par].at[b]],
                rows[par].at[pl.ds(b * L, L)],
                srows[par],
            )

    def drain_gathers(par):
        pltpu.make_async_copy(
            table_ref.at[pl.ds(0, CHUNK * L)], rows[par], srows[par]
        ).wait()

    def fire_idx(ci, par):
        bag0 = base + ci * CHUNK
        pltpu.async_copy(
            text_ref.at[pl.ds(bag0, CHUNK), :], idx[par], sidx[par]
        )

    def drain_idx(par):
        pltpu.make_async_copy(
            text_ref.at[pl.ds(0, CHUNK), :], idx[par], sidx[par]
        ).wait()

    def drain_out(par):
        pltpu.make_async_copy(
            mean[par], out_ref.at[pl.ds(0, CHUNK), :], sout[par]
        ).wait()

    fire_idx(0, 0)
    drain_idx(0)
    fire_gathers(0, 0)
    fire_idx(1, 1)

    def step(ci, par):
        @pl.when(ci + 1 < NCHUNK)
        def _():
            drain_idx(1 - par)
            fire_gathers(ci + 1, 1 - par)

        @pl.when(ci >= 2)
        def _():
            drain_out(par)

        drain_gathers(par)

        @pl.when(ci + 2 < NCHUNK)
        def _():
            fire_idx(ci + 2, par)
        r = rows[par]
        for b in range(CHUNK):
            def body(l, accs):
                return tuple(
                    accs[v] + r[b * L + l, pl.ds(v * 16, 16)]
                    for v in range(NV)
                )
            accs = lax.fori_loop(
                0, L, body,
                tuple(jnp.zeros((16,), jnp.float32) for _ in range(NV)),
                unroll=10,
            )
            for v in range(NV):
                mean[par][b, pl.ds(v * 16, 16)] = accs[v] * (1.0 / L)

        bag0 = base + ci * CHUNK
        pltpu.async_copy(
            mean[par], out_ref.at[pl.ds(bag0, CHUNK), :], sout[par]
        )

    def two_steps(cj, carry):
        step(cj * 2, 0)
        step(cj * 2 + 1, 1)
        return carry

    lax.fori_loop(0, NCHUNK // 2, two_steps, 0)
    drain_out(0)
    drain_out(1)


def _fc_body(x_ref, w_ref, b_ref, o_ref):
    o_ref[...] = lax.dot_general(
        x_ref[...], w_ref[...],
        dimension_numbers=(((1,), (1,)), ((), ())),
        preferred_element_type=jnp.float32,
    ) + b_ref[...]


def _fc(x, w, b2d):
    bm = 1024
    return pl.pallas_call(
        _fc_body,
        grid=(B // bm,),
        in_specs=[
            pl.BlockSpec((bm, D), lambda i: (i, 0)),
            pl.BlockSpec((C, D), lambda i: (0, 0)),
            pl.BlockSpec((1, C), lambda i: (0, 0)),
        ],
        out_specs=pl.BlockSpec((bm, C), lambda i: (i, 0)),
        out_shape=jax.ShapeDtypeStruct((B, C), jnp.float32),
    )(x, w, b2d)


def kernel(text, emb_weight, fc_w, fc_b):
    pooled = _sc_embed(text.astype(jnp.int32), emb_weight)
    return _fc(pooled, fc_w, fc_b.reshape(1, C))
